# baseline (speedup 1.0000x reference)
import functools

import jax
import jax.numpy as jnp
from jax import lax
from jax.experimental import pallas as pl
from jax.experimental.pallas import tpu as pltpu
from jax.experimental.pallas import tpu_sc as plsc

NC = 2
NS = 16
NW = NC * NS
CH = 128

_f32 = jnp.float32
_i32 = jnp.int32


def _sc_mesh():
    return plsc.VectorSubcoreMesh(core_axis_name="c", subcore_axis_name="s")


def _worker_id():
    return lax.axis_index("s") * NC + lax.axis_index("c")


def _zero_rows(ref, nrows, ncols):
    z = jnp.zeros((16,), _f32)

    def body(r, carry):
        for j in range(ncols // 16):
            ref[r, pl.ds(j * 16, 16)] = z
        return carry

    lax.fori_loop(0, nrows, body, 0)


SEG = 4


def _seg_scratch(d, n_pad):
    return [
        pltpu.VMEM((SEG * CH,), _i32),
        pltpu.VMEM((SEG * CH,), _i32),
        pltpu.VMEM((SEG, 1, CH), _i32),
        pltpu.VMEM((SEG, 1, CH), _i32),
        pltpu.VMEM((CH, d), _f32),
        pltpu.VMEM((CH, d), _f32),
        pltpu.VMEM_SHARED((n_pad, d), _f32),
        pltpu.SemaphoreType.DMA,
        pltpu.SemaphoreType.DMA,
        pltpu.SemaphoreType.DMA,
        pltpu.SemaphoreType.DMA,
        pltpu.SemaphoreType.DMA,
        pltpu.SemaphoreType.DMA,
    ]


def _seg_phase(table, srcp, dst3, out, scr, E, n_pad, d, cid, sid, wid):
    (sx0, sx1, dx0, dx1, rows0, rows1, acc,
     isem0, isem1, gsem0, gsem1, ssem0, ssem1) = scr
    rows = (rows0, rows1)
    sxb = (sx0, sx1)
    dxb = (dx0, dx1)
    isem = (isem0, isem1)
    gsem = (gsem0, gsem1)
    ssem = (ssem0, ssem1)
    n_chunks = E // CH
    assert n_chunks * CH == E
    base, extra = divmod(n_chunks, NW)
    assert base // SEG >= 2
    rpt = n_pad // NS
    assert rpt % CH == 0

    sc0 = wid * base + jnp.minimum(wid, extra)
    n_my = base + jnp.where(wid < extra, 1, 0)
    nseg = (n_my + SEG - 1) // SEG

    def preload(ss_val, sb):
        pltpu.async_copy(
            srcp.at[pl.ds((sc0 + ss_val * SEG) * CH, SEG * CH)],
            sxb[sb], isem[sb])
        pltpu.async_copy(dst3.at[pl.ds(sc0 + ss_val * SEG, SEG)],
                         dxb[sb], isem[sb])

    preload(0, 0)
    preload(1, 1)

    _zero_rows(rows0, CH, d)
    r0 = sid * rpt
    for j in range(rpt // CH):
        pltpu.async_copy(rows0, acc.at[pl.ds(r0 + j * CH, CH)], ssem0)
    for j in range(rpt // CH):
        pltpu.make_async_copy(table.at[pl.ds(0, CH)], rows0, ssem0).wait()
    plsc.subcore_barrier()

    def seg_body(ss_val, sb):
        pltpu.make_async_copy(srcp.at[pl.ds(0, SEG * CH)],
                              sxb[sb], isem[sb]).wait()
        pltpu.make_async_copy(dst3.at[pl.ds(0, SEG)],
                              dxb[sb], isem[sb]).wait()
        n_loc = jnp.minimum(n_my - ss_val * SEG, SEG)

        @pl.when(ss_val >= 1)
        def _():
            pltpu.make_async_copy(table.at[pl.ds(0, CH)], rows0,
                                  ssem0).wait()
        pltpu.async_copy(table.at[sxb[sb].at[pl.ds(0, CH)]], rows0,
                         gsem0)
        for g in range(SEG):
            b, ob = g % 2, 1 - g % 2
            if g + 1 < SEG:
                @pl.when(g + 1 < n_loc)
                def _(g=g, b=b, ob=ob):
                    if g >= 1:
                        pltpu.make_async_copy(table.at[pl.ds(0, CH)],
                                              rows[ob], ssem[ob]).wait()
                    else:
                        @pl.when(ss_val >= 1)
                        def _():
                            pltpu.make_async_copy(
                                table.at[pl.ds(0, CH)], rows1,
                                ssem1).wait()

                        @pl.when(jnp.logical_and(ss_val >= 1,
                                                 ss_val + 1 < nseg))
                        def _():
                            preload(ss_val + 1, 1 - sb)
                    pltpu.async_copy(
                        table.at[sxb[sb].at[pl.ds((g + 1) * CH, CH)]],
                        rows[ob], gsem[ob])

            @pl.when(g < n_loc)
            def _(g=g, b=b):
                pltpu.make_async_copy(table.at[pl.ds(0, CH)],
                                      rows[b], gsem[b]).wait()
                pltpu.async_copy(rows[b], acc.at[dxb[sb].at[g, 0]],
                                 ssem[b], add=True)

    def pair_body(sp, carry):
        for sb in (0, 1):
            ss_val = sp * 2 + sb

            @pl.when(ss_val < nseg)
            def _(ss_val=ss_val, sb=sb):
                seg_body(ss_val, sb)
        return carry

    lax.fori_loop(0, (nseg + 1) // 2, pair_body, 0)
    pltpu.make_async_copy(table.at[pl.ds(0, CH)], rows0, ssem0).wait()
    pltpu.make_async_copy(table.at[pl.ds(0, CH)], rows1, ssem1).wait()
    plsc.subcore_barrier()
    pltpu.sync_copy(acc.at[pl.ds(r0, rpt)],
                    out.at[pl.ds(cid * n_pad + r0, rpt)])


def _n_pad(n_dst):
    return ((n_dst + NS * CH - 1) // (NS * CH)) * (NS * CH)


def _seg_pad_chunks(E):
    n_chunks = E // CH
    base, extra = divmod(n_chunks, NW)
    reach = 0
    for t in range(NW):
        n_my = base + (1 if t < extra else 0)
        sc0 = t * base + min(t, extra)
        nseg = -(-n_my // SEG)
        reach = max(reach, sc0 + nseg * SEG)
    return max(0, reach - n_chunks)


@functools.cache
def _make_segsum(E, n_dst, d):
    n_pad = _n_pad(n_dst)

    @functools.partial(
        pl.kernel,
        out_type=jax.ShapeDtypeStruct((NC * n_pad, d), _f32),
        mesh=_sc_mesh(),
        scratch_types=_seg_scratch(d, n_pad),
    )
    def k(table, srcp, dst3, out, *scr):
        cid = lax.axis_index("c")
        sid = lax.axis_index("s")
        wid = sid * NC + cid
        _seg_phase(table, srcp, dst3, out, scr, E, n_pad, d, cid, sid, wid)

    return k, n_pad


@functools.cache
def _make_segsum2(E1, E2, n_dst, d):
    n_pad = _n_pad(n_dst)

    @functools.partial(
        pl.kernel,
        out_type=(jax.ShapeDtypeStruct((NC * n_pad, d), _f32),
                  jax.ShapeDtypeStruct((NC * n_pad, d), _f32)),
        mesh=_sc_mesh(),
        scratch_types=_seg_scratch(d, n_pad),
    )
    def k(tbl1, src1, dst1, tbl2, src2, dst2, out1, out2, *scr):
        cid = lax.axis_index("c")
        sid = lax.axis_index("s")
        wid = sid * NC + cid
        _seg_phase(tbl1, src1, dst1, out1, scr, E1, n_pad, d, cid, sid, wid)
        _seg_phase(tbl2, src2, dst2, out2, scr, E2, n_pad, d, cid, sid, wid)

    return k, n_pad


@functools.cache
def _make_prep(d, B, E_a, n_a, E_p, n_p):
    CG = 32
    n_chunks = B // CG
    mpt = n_chunks // NW
    assert mpt * NW == n_chunks and mpt >= 2
    apt = E_a // NW
    assert apt * NW == E_a and apt % 16 == 0
    n_grp = E_p // 16
    assert n_grp * 16 == E_p
    gbase, gextra = divmod(n_grp, NW)
    wlen = (gbase + 1) * 16

    @functools.partial(
        pl.kernel,
        out_type=(jax.ShapeDtypeStruct((B, d), _f32),
                  jax.ShapeDtypeStruct((NW, 1, n_a), _f32),
                  jax.ShapeDtypeStruct((NW, 1, n_p), _f32)),
        mesh=_sc_mesh(),
        scratch_types=[
            pltpu.VMEM((mpt * CG,), _i32),
            pltpu.VMEM((CG, d), _f32),
            pltpu.VMEM((CG, d), _f32),
            pltpu.VMEM((apt,), _i32),
            pltpu.VMEM((wlen,), _i32),
            pltpu.VMEM((1, n_a), _f32),
            pltpu.VMEM((1, n_p), _f32),
            pltpu.SemaphoreType.DMA,
            pltpu.SemaphoreType.DMA,
            pltpu.SemaphoreType.DMA,
            pltpu.SemaphoreType.DMA,
            pltpu.SemaphoreType.DMA,
        ],
        compiler_params=pltpu.CompilerParams(needs_layout_passes=False),
    )
    def k(table, idx, dst_a, dst_p, cls, deg_out, cnt_out,
          idxall, rows0, rows1, abuf, wbuf, ha, hp,
          hsem, gsem0, gsem1, ssem0, ssem1):
        wid = _worker_id()
        rows = (rows0, rows1)
        gsem = (gsem0, gsem1)
        ssem = (ssem0, ssem1)

        pltpu.async_copy(dst_a.at[pl.ds(wid * apt, apt)], abuf, hsem)
        g0 = wid * gbase + jnp.minimum(wid, gextra)
        pltpu.async_copy(dst_p.at[pl.ds(g0 * 16, wlen)], wbuf, hsem)

        c0 = wid * mpt
        pltpu.sync_copy(idx.at[pl.ds(c0 * CG, mpt * CG)], idxall)
        pltpu.async_copy(table.at[idxall.at[pl.ds(0, CG)]], rows0, gsem0)

        z = jnp.zeros((16,), _f32)

        def za(i, carry):
            ha[0, pl.ds(i * 16, 16)] = z
            return carry

        lax.fori_loop(0, n_a // 16, za, 0)

        def zp(i, carry):
            hp[0, pl.ds(i * 16, 16)] = z
            return carry

        lax.fori_loop(0, n_p // 16, zp, 0)

        for g in range(mpt):
            b, ob = g % 2, 1 - g % 2
            if g + 1 < mpt:
                if g >= 1:
                    pltpu.make_async_copy(
                        rows[ob], cls.at[pl.ds((c0 + g - 1) * CG, CG)],
                        ssem[ob]).wait()
                pltpu.async_copy(
                    table.at[idxall.at[pl.ds((g + 1) * CG, CG)]],
                    rows[ob], gsem[ob])
            pltpu.make_async_copy(table.at[pl.ds(0, CG)], rows[b],
                                  gsem[b]).wait()
            pltpu.async_copy(rows[b], cls.at[pl.ds((c0 + g) * CG, CG)],
                             ssem[b])
        for g in (mpt - 2, mpt - 1):
            pltpu.make_async_copy(rows[g % 2],
                                  cls.at[pl.ds((c0 + g) * CG, CG)],
                                  ssem[g % 2]).wait()

        pltpu.make_async_copy(dst_a.at[pl.ds(0, apt)], abuf, hsem).wait()
        pltpu.make_async_copy(dst_a.at[pl.ds(0, wlen)], wbuf, hsem).wait()
        ones = jnp.ones((16,), _f32)
        zi = jnp.zeros((16,), _i32)

        def abody(i, carry):
            idxv = abuf[pl.ds(i * 16, 16)]
            plsc.addupdate_scatter(ha, [zi, idxv], ones)
            return carry

        lax.fori_loop(0, apt // 16, abody, 0)
        n_g = gbase + jnp.where(wid < gextra, 1, 0)

        def pbody(i, carry):
            idxv = wbuf[pl.ds(i * 16, 16)]
            plsc.addupdate_scatter(hp, [zi, idxv], ones)
            return carry

        lax.fori_loop(0, n_g, pbody, 0)
        pltpu.sync_copy(ha, deg_out.at[wid])
        pltpu.sync_copy(hp, cnt_out.at[wid])

    return k


def _mxu(a, b):
    return jnp.dot(a.astype(jnp.bfloat16), b.astype(jnp.bfloat16),
                   preferred_element_type=_f32)


_R = 1000


def _pre_gcn(x, W, degp):
    n, h = x.shape

    def body(x_ref, w_ref, dp_ref, g_ref):
        deg = jnp.sum(dp_ref[...], axis=1) + 1.0
        dinv = lax.rsqrt(deg)
        g_ref[...] = _mxu(x_ref[...], w_ref[...]) * dinv[:, None]

    return pl.pallas_call(
        body,
        grid=(n // _R,),
        in_specs=[
            pl.BlockSpec((_R, h), lambda i: (i, 0)),
            pl.BlockSpec((h, h), lambda i: (0, 0)),
            pl.BlockSpec((_R, NW), lambda i: (i, 0)),
        ],
        out_specs=pl.BlockSpec((_R, h), lambda i: (i, 0)),
        out_shape=jax.ShapeDtypeStruct((n, h), _f32),
    )(x, W, degp)


def _post_gcn(S, g, degp, b):
    n, h = g.shape

    def body(s_ref, g_ref, dp_ref, b_ref, o_ref):
        deg = jnp.sum(dp_ref[...], axis=1) + 1.0
        dinv = lax.rsqrt(deg)
        s = s_ref[0] + s_ref[1] + g_ref[...]
        o_ref[...] = jnp.maximum(s * dinv[:, None] + b_ref[...], 0.0)

    return pl.pallas_call(
        body,
        grid=(n // _R,),
        in_specs=[
            pl.BlockSpec((NC, _R, h), lambda i: (0, i, 0)),
            pl.BlockSpec((_R, h), lambda i: (i, 0)),
            pl.BlockSpec((_R, NW), lambda i: (i, 0)),
            pl.BlockSpec((1, h), lambda i: (0, 0)),
        ],
        out_specs=pl.BlockSpec((_R, h), lambda i: (i, 0)),
        out_shape=jax.ShapeDtypeStruct((n, h), _f32),
    )(S, g, degp, b)


def _paper1(cls_emb, poolW, poolb, S1, cntp, Wl, Wrb, Wrf, featp, b1):
    n, db = cls_emb.shape
    h = Wl.shape[0]
    df = featp.shape[1]

    def body(c_ref, pw_ref, pb_ref, s_ref, ct_ref, wl_ref, wb_ref, wf_ref,
             f_ref, b_ref, o_ref):
        pooled = jnp.tanh(_mxu(c_ref[...], pw_ref[...]) + pb_ref[...])
        cnt = jnp.sum(ct_ref[...], axis=1)
        inv = 1.0 / jnp.maximum(cnt, 1.0)
        mean = (s_ref[0] + s_ref[1]) * inv[:, None]
        o = _mxu(mean, wl_ref[...])
        o = o + _mxu(pooled, wb_ref[...])
        o = o + _mxu(f_ref[...], wf_ref[...])
        o_ref[...] = jnp.maximum(o + b_ref[...], 0.0)

    return pl.pallas_call(
        body,
        grid=(n // _R,),
        in_specs=[
            pl.BlockSpec((_R, db), lambda i: (i, 0)),
            pl.BlockSpec((db, db), lambda i: (0, 0)),
            pl.BlockSpec((1, db), lambda i: (0, 0)),
            pl.BlockSpec((NC, _R, h), lambda i: (0, i, 0)),
            pl.BlockSpec((_R, NW), lambda i: (i, 0)),
            pl.BlockSpec((h, h), lambda i: (0, 0)),
            pl.BlockSpec((db, h), lambda i: (0, 0)),
            pl.BlockSpec((df, h), lambda i: (0, 0)),
            pl.BlockSpec((_R, df), lambda i: (i, 0)),
            pl.BlockSpec((1, h), lambda i: (0, 0)),
        ],
        out_specs=pl.BlockSpec((_R, h), lambda i: (i, 0)),
        out_shape=jax.ShapeDtypeStruct((n, h), _f32),
    )(cls_emb, poolW, poolb, S1, cntp, Wl, Wrb, Wrf, featp, b1)


def _paper2(S2, cntp, p1, Wl, Wr, b2, linW, linb):
    n, h = p1.shape

    def body(s_ref, ct_ref, p_ref, wl_ref, wr_ref, b_ref, lw_ref, lb_ref,
             o_ref):
        cnt = jnp.sum(ct_ref[...], axis=1)
        inv = 1.0 / jnp.maximum(cnt, 1.0)
        mean = (s_ref[0] + s_ref[1]) * inv[:, None]
        p2 = jnp.maximum(
            _mxu(mean, wl_ref[...])
            + _mxu(p_ref[...], wr_ref[...])
            + b_ref[...], 0.0)
        o_ref[...] = _mxu(p2, lw_ref[...]) + lb_ref[...]

    return pl.pallas_call(
        body,
        grid=(n // _R,),
        in_specs=[
            pl.BlockSpec((NC, _R, h), lambda i: (0, i, 0)),
            pl.BlockSpec((_R, NW), lambda i: (i, 0)),
            pl.BlockSpec((_R, h), lambda i: (i, 0)),
            pl.BlockSpec((h, h), lambda i: (0, 0)),
            pl.BlockSpec((h, h), lambda i: (0, 0)),
            pl.BlockSpec((1, h), lambda i: (0, 0)),
            pl.BlockSpec((h, h), lambda i: (0, 0)),
            pl.BlockSpec((1, h), lambda i: (0, 0)),
        ],
        out_specs=pl.BlockSpec((_R, h), lambda i: (i, 0)),
        out_shape=jax.ShapeDtypeStruct((n, h), _f32),
    )(S2, cntp, p1, Wl, Wr, b2, linW, linb)


def kernel(x_author, paper_tokens, paper_feat, edge_index_aa,
           edge_index_writes, scibert_emb, pool_W, pool_b, gcn1_W, gcn1_b,
           sage1_Wl, sage1_Wr, sage1_b, gcn2_W, gcn2_b, sage2_Wl, sage2_Wr,
           sage2_b, lin_W, lin_b):
    n_a, h = x_author.shape
    n_p, d_feat = paper_feat.shape
    d_bert = scibert_emb.shape[1]
    out_dim = lin_W.shape[1]

    src_aa = edge_index_aa[0].astype(_i32)
    dst_aa = edge_index_aa[1].astype(_i32)
    src_wr = edge_index_writes[0].astype(_i32)
    dst_wr = edge_index_writes[1].astype(_i32)
    e_aa = src_aa.shape[0]
    e_wr = src_wr.shape[0]

    def padded(a, n_extra_chunks):
        if n_extra_chunks == 0:
            return a
        return jnp.concatenate([a, jnp.zeros((n_extra_chunks * CH,), _i32)])

    src_aa_p = padded(src_aa, _seg_pad_chunks(e_aa))
    dst_aa_3 = padded(dst_aa, _seg_pad_chunks(e_aa)).reshape(-1, 1, CH)
    src_wr_p = padded(src_wr, _seg_pad_chunks(e_wr))
    dst_wr_p = padded(dst_wr, max(_seg_pad_chunks(e_wr), 1))
    dst_wr_3 = dst_wr_p[:(e_wr // CH + _seg_pad_chunks(e_wr)) * CH
                        ].reshape(-1, 1, CH)
    cls_idx = paper_tokens[:, 0].astype(_i32)
    b_pad = ((n_p + 32 * NW - 1) // (32 * NW)) * (32 * NW)
    cls_idx = jnp.concatenate([cls_idx, jnp.zeros((b_pad - n_p,), _i32)])

    cls_rows, degp, cntp = _make_prep(d_bert, b_pad, e_aa, n_a, e_wr, n_p)(
        scibert_emb, cls_idx, dst_aa, dst_wr_p)
    cls_rows = cls_rows[:n_p]
    degp = degp.reshape(NW, n_a).T
    cntp = cntp.reshape(NW, n_p).T

    g = _pre_gcn(x_author, gcn1_W, degp)

    assert n_a == n_p
    seg2, npad_p = _make_segsum2(e_wr, e_aa, n_p, h)
    npad_a = npad_p
    S1, SA = seg2(x_author, src_wr_p, dst_wr_3, g, src_aa_p, dst_aa_3)
    S1 = S1.reshape(NC, npad_p, h)
    SA = SA.reshape(NC, npad_a, h)
    a1 = _post_gcn(SA, g, degp, gcn1_b.reshape(1, h))
    seg_wr, _ = _make_segsum(e_wr, n_p, h)

    Wrb = sage1_Wr[:d_bert]
    Wrf = sage1_Wr[d_bert:]
    p1 = _paper1(cls_rows, pool_W, pool_b.reshape(1, d_bert), S1, cntp,
                 sage1_Wl, Wrb, Wrf, paper_feat, sage1_b.reshape(1, h))

    S2 = seg_wr(a1, src_wr_p, dst_wr_3).reshape(NC, npad_p, h)
    linWp = jnp.pad(lin_W, ((0, 0), (0, h - out_dim)))
    linbp = jnp.pad(lin_b, (0, h - out_dim)).reshape(1, h)
    out = _paper2(S2, cntp, p1, sage2_Wl, sage2_Wr,
                  sage2_b.reshape(1, h), linWp, linbp)
    return out[:, :out_dim]


# speedup vs baseline: 1.0229x; 1.0229x over previous
import functools

import jax
import jax.numpy as jnp
from jax import lax
from jax.experimental import pallas as pl
from jax.experimental.pallas import tpu as pltpu
from jax.experimental.pallas import tpu_sc as plsc

NC = 2
NS = 16
NW = NC * NS
CH = 128

_f32 = jnp.float32
_i32 = jnp.int32


def _sc_mesh():
    return plsc.VectorSubcoreMesh(core_axis_name="c", subcore_axis_name="s")


def _worker_id():
    return lax.axis_index("s") * NC + lax.axis_index("c")


def _zero_rows(ref, nrows, ncols):
    z = jnp.zeros((16,), _f32)

    def body(r, carry):
        for j in range(ncols // 16):
            ref[r, pl.ds(j * 16, 16)] = z
        return carry

    lax.fori_loop(0, nrows, body, 0)


SEG = 4


def _seg_scratch(d, n_pad):
    return [
        pltpu.VMEM((SEG * CH,), _i32),
        pltpu.VMEM((SEG * CH,), _i32),
        pltpu.VMEM((SEG, 1, CH), _i32),
        pltpu.VMEM((SEG, 1, CH), _i32),
        pltpu.VMEM((CH, d), _f32),
        pltpu.VMEM((CH, d), _f32),
        pltpu.VMEM_SHARED((n_pad, d), _f32),
        pltpu.SemaphoreType.DMA,
        pltpu.SemaphoreType.DMA,
        pltpu.SemaphoreType.DMA,
        pltpu.SemaphoreType.DMA,
        pltpu.SemaphoreType.DMA,
        pltpu.SemaphoreType.DMA,
    ]


def _seg_phase(table, srcp, dst3, out, scr, E, n_pad, d, cid, sid, wid):
    (sx0, sx1, dx0, dx1, rows0, rows1, acc,
     isem0, isem1, gsem0, gsem1, ssem0, ssem1) = scr
    rows = (rows0, rows1)
    sxb = (sx0, sx1)
    dxb = (dx0, dx1)
    isem = (isem0, isem1)
    gsem = (gsem0, gsem1)
    ssem = (ssem0, ssem1)
    n_chunks = E // CH
    assert n_chunks * CH == E
    base, extra = divmod(n_chunks, NW)
    assert base // SEG >= 2
    rpt = n_pad // NS
    assert rpt % CH == 0

    sc0 = wid * base + jnp.minimum(wid, extra)
    n_my = base + jnp.where(wid < extra, 1, 0)
    nseg = (n_my + SEG - 1) // SEG

    def preload(ss_val, sb):
        pltpu.async_copy(
            srcp.at[pl.ds((sc0 + ss_val * SEG) * CH, SEG * CH)],
            sxb[sb], isem[sb])
        pltpu.async_copy(dst3.at[pl.ds(sc0 + ss_val * SEG, SEG)],
                         dxb[sb], isem[sb])

    preload(0, 0)
    preload(1, 1)

    _zero_rows(rows0, CH, d)
    r0 = sid * rpt
    for j in range(rpt // CH):
        pltpu.async_copy(rows0, acc.at[pl.ds(r0 + j * CH, CH)], ssem0)
    for j in range(rpt // CH):
        pltpu.make_async_copy(table.at[pl.ds(0, CH)], rows0, ssem0).wait()
    plsc.subcore_barrier()

    def seg_body(ss_val, sb):
        pltpu.make_async_copy(srcp.at[pl.ds(0, SEG * CH)],
                              sxb[sb], isem[sb]).wait()
        pltpu.make_async_copy(dst3.at[pl.ds(0, SEG)],
                              dxb[sb], isem[sb]).wait()
        n_loc = jnp.minimum(n_my - ss_val * SEG, SEG)

        @pl.when(ss_val >= 1)
        def _():
            pltpu.make_async_copy(table.at[pl.ds(0, CH)], rows0,
                                  ssem0).wait()
        pltpu.async_copy(table.at[sxb[sb].at[pl.ds(0, CH)]], rows0,
                         gsem0)
        for g in range(SEG):
            b, ob = g % 2, 1 - g % 2
            if g + 1 < SEG:
                @pl.when(g + 1 < n_loc)
                def _(g=g, b=b, ob=ob):
                    if g >= 1:
                        pltpu.make_async_copy(table.at[pl.ds(0, CH)],
                                              rows[ob], ssem[ob]).wait()
                    else:
                        @pl.when(ss_val >= 1)
                        def _():
                            pltpu.make_async_copy(
                                table.at[pl.ds(0, CH)], rows1,
                                ssem1).wait()

                        @pl.when(jnp.logical_and(ss_val >= 1,
                                                 ss_val + 1 < nseg))
                        def _():
                            preload(ss_val + 1, 1 - sb)
                    pltpu.async_copy(
                        table.at[sxb[sb].at[pl.ds((g + 1) * CH, CH)]],
                        rows[ob], gsem[ob])

            @pl.when(g < n_loc)
            def _(g=g, b=b):
                pltpu.make_async_copy(table.at[pl.ds(0, CH)],
                                      rows[b], gsem[b]).wait()
                pltpu.async_copy(rows[b], acc.at[dxb[sb].at[g, 0]],
                                 ssem[b], add=True)

    def pair_body(sp, carry):
        for sb in (0, 1):
            ss_val = sp * 2 + sb

            @pl.when(ss_val < nseg)
            def _(ss_val=ss_val, sb=sb):
                seg_body(ss_val, sb)
        return carry

    lax.fori_loop(0, (nseg + 1) // 2, pair_body, 0)
    pltpu.make_async_copy(table.at[pl.ds(0, CH)], rows0, ssem0).wait()
    pltpu.make_async_copy(table.at[pl.ds(0, CH)], rows1, ssem1).wait()
    plsc.subcore_barrier()
    pltpu.sync_copy(acc.at[pl.ds(r0, rpt)],
                    out.at[pl.ds(cid * n_pad + r0, rpt)])


def _n_pad(n_dst):
    return ((n_dst + NS * CH - 1) // (NS * CH)) * (NS * CH)


def _seg_pad_chunks(E):
    n_chunks = E // CH
    base, extra = divmod(n_chunks, NW)
    reach = 0
    for t in range(NW):
        n_my = base + (1 if t < extra else 0)
        sc0 = t * base + min(t, extra)
        nseg = -(-n_my // SEG)
        reach = max(reach, sc0 + nseg * SEG)
    return max(0, reach - n_chunks)


@functools.cache
def _make_segsum(E, n_dst, d):
    n_pad = _n_pad(n_dst)

    @functools.partial(
        pl.kernel,
        out_type=jax.ShapeDtypeStruct((NC * n_pad, d), _f32),
        mesh=_sc_mesh(),
        scratch_types=_seg_scratch(d, n_pad),
    )
    def k(table, srcp, dst3, out, *scr):
        cid = lax.axis_index("c")
        sid = lax.axis_index("s")
        wid = sid * NC + cid
        _seg_phase(table, srcp, dst3, out, scr, E, n_pad, d, cid, sid, wid)

    return k, n_pad


@functools.cache
def _make_segsum2(E1, E2, n_dst, d):
    n_pad = _n_pad(n_dst)

    @functools.partial(
        pl.kernel,
        out_type=(jax.ShapeDtypeStruct((NC * n_pad, d), _f32),
                  jax.ShapeDtypeStruct((NC * n_pad, d), _f32)),
        mesh=_sc_mesh(),
        scratch_types=_seg_scratch(d, n_pad),
    )
    def k(tbl1, src1, dst1, tbl2, src2, dst2, out1, out2, *scr):
        cid = lax.axis_index("c")
        sid = lax.axis_index("s")
        wid = sid * NC + cid
        _seg_phase(tbl1, src1, dst1, out1, scr, E1, n_pad, d, cid, sid, wid)
        _seg_phase(tbl2, src2, dst2, out2, scr, E2, n_pad, d, cid, sid, wid)

    return k, n_pad


@functools.cache
def _make_prep(d, B, E_a, n_a, E_p, n_p):
    CG = 32
    n_chunks = B // CG
    mpt = n_chunks // NW
    assert mpt * NW == n_chunks and mpt >= 2
    apt = E_a // NW
    assert apt * NW == E_a and apt % 16 == 0
    n_grp = E_p // 16
    assert n_grp * 16 == E_p
    gbase, gextra = divmod(n_grp, NW)
    wlen = (gbase + 1) * 16

    @functools.partial(
        pl.kernel,
        out_type=(jax.ShapeDtypeStruct((B, d), _f32),
                  jax.ShapeDtypeStruct((NW, 1, n_a), _f32),
                  jax.ShapeDtypeStruct((NW, 1, n_p), _f32)),
        mesh=_sc_mesh(),
        scratch_types=[
            pltpu.VMEM((mpt * CG,), _i32),
            pltpu.VMEM((CG, d), _f32),
            pltpu.VMEM((CG, d), _f32),
            pltpu.VMEM((apt,), _i32),
            pltpu.VMEM((wlen,), _i32),
            pltpu.VMEM((1, n_a), _f32),
            pltpu.VMEM((1, n_p), _f32),
            pltpu.SemaphoreType.DMA,
            pltpu.SemaphoreType.DMA,
            pltpu.SemaphoreType.DMA,
            pltpu.SemaphoreType.DMA,
            pltpu.SemaphoreType.DMA,
        ],
        compiler_params=pltpu.CompilerParams(needs_layout_passes=False),
    )
    def k(table, idx, dst_a, dst_p, cls, deg_out, cnt_out,
          idxall, rows0, rows1, abuf, wbuf, ha, hp,
          hsem, gsem0, gsem1, ssem0, ssem1):
        wid = _worker_id()
        rows = (rows0, rows1)
        gsem = (gsem0, gsem1)
        ssem = (ssem0, ssem1)

        pltpu.async_copy(dst_a.at[pl.ds(wid * apt, apt)], abuf, hsem)
        g0 = wid * gbase + jnp.minimum(wid, gextra)
        pltpu.async_copy(dst_p.at[pl.ds(g0 * 16, wlen)], wbuf, hsem)

        c0 = wid * mpt
        pltpu.sync_copy(idx.at[pl.ds(c0 * CG, mpt * CG)], idxall)
        pltpu.async_copy(table.at[idxall.at[pl.ds(0, CG)]], rows0, gsem0)

        z = jnp.zeros((16,), _f32)

        def za(i, carry):
            ha[0, pl.ds(i * 16, 16)] = z
            return carry

        lax.fori_loop(0, n_a // 16, za, 0)

        def zp(i, carry):
            hp[0, pl.ds(i * 16, 16)] = z
            return carry

        lax.fori_loop(0, n_p // 16, zp, 0)

        for g in range(mpt):
            b, ob = g % 2, 1 - g % 2
            if g + 1 < mpt:
                if g >= 1:
                    pltpu.make_async_copy(
                        rows[ob], cls.at[pl.ds((c0 + g - 1) * CG, CG)],
                        ssem[ob]).wait()
                pltpu.async_copy(
                    table.at[idxall.at[pl.ds((g + 1) * CG, CG)]],
                    rows[ob], gsem[ob])
            pltpu.make_async_copy(table.at[pl.ds(0, CG)], rows[b],
                                  gsem[b]).wait()
            pltpu.async_copy(rows[b], cls.at[pl.ds((c0 + g) * CG, CG)],
                             ssem[b])
        for g in (mpt - 2, mpt - 1):
            pltpu.make_async_copy(rows[g % 2],
                                  cls.at[pl.ds((c0 + g) * CG, CG)],
                                  ssem[g % 2]).wait()

        pltpu.make_async_copy(dst_a.at[pl.ds(0, apt)], abuf, hsem).wait()
        pltpu.make_async_copy(dst_a.at[pl.ds(0, wlen)], wbuf, hsem).wait()
        ones = jnp.ones((16,), _f32)
        zi = jnp.zeros((16,), _i32)

        def abody(i, carry):
            idxv = abuf[pl.ds(i * 16, 16)]
            plsc.addupdate_scatter(ha, [zi, idxv], ones)
            return carry

        lax.fori_loop(0, apt // 16, abody, 0)
        n_g = gbase + jnp.where(wid < gextra, 1, 0)

        def pbody(i, carry):
            idxv = wbuf[pl.ds(i * 16, 16)]
            plsc.addupdate_scatter(hp, [zi, idxv], ones)
            return carry

        lax.fori_loop(0, n_g, pbody, 0)
        pltpu.sync_copy(ha, deg_out.at[wid])
        pltpu.sync_copy(hp, cnt_out.at[wid])

    return k


def _mxu(a, b):
    return jnp.dot(a.astype(jnp.bfloat16), b.astype(jnp.bfloat16),
                   preferred_element_type=_f32)


_R = 1000


def _pre_gcn(x, W, degp):
    n, h = x.shape

    def body(x_ref, w_ref, dp_ref, g_ref):
        deg = jnp.sum(dp_ref[...], axis=1) + 1.0
        dinv = lax.rsqrt(deg)
        g_ref[...] = _mxu(x_ref[...], w_ref[...]) * dinv[:, None]

    return pl.pallas_call(
        body,
        grid=(n // _R,),
        in_specs=[
            pl.BlockSpec((_R, h), lambda i: (i, 0)),
            pl.BlockSpec((h, h), lambda i: (0, 0)),
            pl.BlockSpec((_R, NW), lambda i: (i, 0)),
        ],
        out_specs=pl.BlockSpec((_R, h), lambda i: (i, 0)),
        out_shape=jax.ShapeDtypeStruct((n, h), _f32),
    )(x, W, degp)


def _post_gcn(S, g, degp, b):
    n, h = g.shape

    def body(s_ref, g_ref, dp_ref, b_ref, o_ref):
        deg = jnp.sum(dp_ref[...], axis=1) + 1.0
        dinv = lax.rsqrt(deg)
        s = s_ref[0] + s_ref[1] + g_ref[...]
        o_ref[...] = jnp.maximum(s * dinv[:, None] + b_ref[...], 0.0)

    return pl.pallas_call(
        body,
        grid=(n // _R,),
        in_specs=[
            pl.BlockSpec((NC, _R, h), lambda i: (0, i, 0)),
            pl.BlockSpec((_R, h), lambda i: (i, 0)),
            pl.BlockSpec((_R, NW), lambda i: (i, 0)),
            pl.BlockSpec((1, h), lambda i: (0, 0)),
        ],
        out_specs=pl.BlockSpec((_R, h), lambda i: (i, 0)),
        out_shape=jax.ShapeDtypeStruct((n, h), _f32),
    )(S, g, degp, b)


def _paper1(cls_emb, poolW, poolb, S1, cntp, Wl, Wrb, Wrf, featp, b1):
    n = cntp.shape[0]
    db = cls_emb.shape[1]
    h = Wl.shape[0]
    df = featp.shape[1]

    def body(c_ref, pw_ref, pb_ref, s_ref, ct_ref, wl_ref, wb_ref, wf_ref,
             f_ref, b_ref, o_ref):
        pooled = jnp.tanh(_mxu(c_ref[...], pw_ref[...]) + pb_ref[...])
        cnt = jnp.sum(ct_ref[...], axis=1)
        inv = 1.0 / jnp.maximum(cnt, 1.0)
        mean = (s_ref[0] + s_ref[1]) * inv[:, None]
        o = _mxu(mean, wl_ref[...])
        o = o + _mxu(pooled, wb_ref[...])
        o = o + _mxu(f_ref[...], wf_ref[...])
        o_ref[...] = jnp.maximum(o + b_ref[...], 0.0)

    return pl.pallas_call(
        body,
        grid=(n // _R,),
        in_specs=[
            pl.BlockSpec((_R, db), lambda i: (i, 0)),
            pl.BlockSpec((db, db), lambda i: (0, 0)),
            pl.BlockSpec((1, db), lambda i: (0, 0)),
            pl.BlockSpec((NC, _R, h), lambda i: (0, i, 0)),
            pl.BlockSpec((_R, NW), lambda i: (i, 0)),
            pl.BlockSpec((h, h), lambda i: (0, 0)),
            pl.BlockSpec((db, h), lambda i: (0, 0)),
            pl.BlockSpec((df, h), lambda i: (0, 0)),
            pl.BlockSpec((_R, df), lambda i: (i, 0)),
            pl.BlockSpec((1, h), lambda i: (0, 0)),
        ],
        out_specs=pl.BlockSpec((_R, h), lambda i: (i, 0)),
        out_shape=jax.ShapeDtypeStruct((n, h), _f32),
    )(cls_emb, poolW, poolb, S1, cntp, Wl, Wrb, Wrf, featp, b1)


def _paper2(S2, cntp, p1, Wl, Wr, b2, linW, linb):
    n, h = p1.shape

    def body(s_ref, ct_ref, p_ref, wl_ref, wr_ref, b_ref, lw_ref, lb_ref,
             o_ref):
        cnt = jnp.sum(ct_ref[...], axis=1)
        inv = 1.0 / jnp.maximum(cnt, 1.0)
        mean = (s_ref[0] + s_ref[1]) * inv[:, None]
        p2 = jnp.maximum(
            _mxu(mean, wl_ref[...])
            + _mxu(p_ref[...], wr_ref[...])
            + b_ref[...], 0.0)
        o_ref[...] = _mxu(p2, lw_ref[...]) + lb_ref[...]

    return pl.pallas_call(
        body,
        grid=(n // _R,),
        in_specs=[
            pl.BlockSpec((NC, _R, h), lambda i: (0, i, 0)),
            pl.BlockSpec((_R, NW), lambda i: (i, 0)),
            pl.BlockSpec((_R, h), lambda i: (i, 0)),
            pl.BlockSpec((h, h), lambda i: (0, 0)),
            pl.BlockSpec((h, h), lambda i: (0, 0)),
            pl.BlockSpec((1, h), lambda i: (0, 0)),
            pl.BlockSpec((h, h), lambda i: (0, 0)),
            pl.BlockSpec((1, h), lambda i: (0, 0)),
        ],
        out_specs=pl.BlockSpec((_R, h), lambda i: (i, 0)),
        out_shape=jax.ShapeDtypeStruct((n, h), _f32),
    )(S2, cntp, p1, Wl, Wr, b2, linW, linb)


def kernel(x_author, paper_tokens, paper_feat, edge_index_aa,
           edge_index_writes, scibert_emb, pool_W, pool_b, gcn1_W, gcn1_b,
           sage1_Wl, sage1_Wr, sage1_b, gcn2_W, gcn2_b, sage2_Wl, sage2_Wr,
           sage2_b, lin_W, lin_b):
    n_a, h = x_author.shape
    n_p, d_feat = paper_feat.shape
    d_bert = scibert_emb.shape[1]
    out_dim = lin_W.shape[1]

    src_aa = edge_index_aa[0].astype(_i32)
    dst_aa = edge_index_aa[1].astype(_i32)
    src_wr = edge_index_writes[0].astype(_i32)
    dst_wr = edge_index_writes[1].astype(_i32)
    e_aa = src_aa.shape[0]
    e_wr = src_wr.shape[0]

    def padded(a, n_extra_chunks):
        if n_extra_chunks == 0:
            return a
        return jnp.concatenate([a, jnp.zeros((n_extra_chunks * CH,), _i32)])

    src_aa_p = padded(src_aa, _seg_pad_chunks(e_aa))
    dst_aa_3 = padded(dst_aa, _seg_pad_chunks(e_aa)).reshape(-1, 1, CH)
    src_wr_p = padded(src_wr, _seg_pad_chunks(e_wr))
    dst_wr_p = padded(dst_wr, max(_seg_pad_chunks(e_wr), 1))
    dst_wr_3 = dst_wr_p[:(e_wr // CH + _seg_pad_chunks(e_wr)) * CH
                        ].reshape(-1, 1, CH)
    cls_idx = paper_tokens[:, 0].astype(_i32)
    b_pad = ((n_p + 32 * NW - 1) // (32 * NW)) * (32 * NW)
    cls_idx = jnp.concatenate([cls_idx, jnp.zeros((b_pad - n_p,), _i32)])

    cls_rows, degp, cntp = _make_prep(d_bert, b_pad, e_aa, n_a, e_wr, n_p)(
        scibert_emb, cls_idx, dst_aa, dst_wr_p)
    degp = degp.reshape(NW, n_a).T
    cntp = cntp.reshape(NW, n_p).T

    g = _pre_gcn(x_author, gcn1_W, degp)

    assert n_a == n_p
    seg2, npad_p = _make_segsum2(e_wr, e_aa, n_p, h)
    npad_a = npad_p
    S1, SA = seg2(x_author, src_wr_p, dst_wr_3, g, src_aa_p, dst_aa_3)
    S1 = S1.reshape(NC, npad_p, h)
    SA = SA.reshape(NC, npad_a, h)
    a1 = _post_gcn(SA, g, degp, gcn1_b.reshape(1, h))
    seg_wr, _ = _make_segsum(e_wr, n_p, h)

    Wrb = sage1_Wr[:d_bert]
    Wrf = sage1_Wr[d_bert:]
    p1 = _paper1(cls_rows, pool_W, pool_b.reshape(1, d_bert), S1, cntp,
                 sage1_Wl, Wrb, Wrf, paper_feat, sage1_b.reshape(1, h))

    S2 = seg_wr(a1, src_wr_p, dst_wr_3).reshape(NC, npad_p, h)
    linWp = jnp.pad(lin_W, ((0, 0), (0, h - out_dim)))
    linbp = jnp.pad(lin_b, (0, h - out_dim)).reshape(1, h)
    out = _paper2(S2, cntp, p1, sage2_Wl, sage2_Wr,
                  sage2_b.reshape(1, h), linWp, linbp)
    return out[:, :out_dim]
